# baseline (speedup 1.0000x reference)
import jax
import jax.numpy as jnp
from jax.experimental import pallas as pl

K = 256
BOX_SIZE = 32.0
ROWS_PER_STEP = 4


def _matvec_body(x_ref, w_ref, o_ref):
    wv = w_ref[...]
    for r in range(x_ref.shape[0]):
        o_ref[r] = jnp.dot(wv, x_ref[r], preferred_element_type=jnp.float32)


def kernel(f8, w, b, image_height, image_width):
    B, V, C, H, W = f8.shape
    HW = H * W
    BV = B * V
    x = f8.reshape(BV, C, HW)
    R = ROWS_PER_STEP
    logits = pl.pallas_call(
        _matvec_body,
        grid=(BV // R,),
        in_specs=[
            pl.BlockSpec((R, C, HW), lambda i: (i, 0, 0)),
            pl.BlockSpec((1, C), lambda i: (0, 0)),
        ],
        out_specs=pl.BlockSpec((R, 1, HW), lambda i: (i, 0, 0)),
        out_shape=jax.ShapeDtypeStruct((BV, 1, HW), jnp.float32),
    )(x, w.reshape(1, C))

    scores = jax.nn.sigmoid(logits.reshape(B, V, HW) + b)
    top_values, top_idx = jax.lax.top_k(scores, K)
    ys = (top_idx // W).astype(jnp.float32) * (image_height / H)
    xs = (top_idx % W).astype(jnp.float32) * (image_width / W)
    half = BOX_SIZE * 0.5
    boxes = jnp.stack((xs - half, ys - half, xs + half, ys + half), axis=-1)
    return boxes, top_values
